# Initial kernel scaffold; baseline (speedup 1.0000x reference)
import jax
import jax.numpy as jnp
from jax.experimental import pallas as pl


def kernel(x_a, x_b, edge_index_ab, edge_index_ba, W_in, b_in, W_taps, b_taps, gamma, beta, W_out, b_out):
    raise NotImplementedError("write your pallas kernel here")


# speedup vs baseline: 1.0037x; 1.0037x over previous
import jax
import jax.numpy as jnp
from jax.experimental import pallas as pl

N = 50000
D_H = 32
N_TAPS = 4
N_LAYERS = 2


def _mm_body(x_ref, w_ref, b_ref, o_ref):
    o_ref[...] = jnp.dot(x_ref[...], w_ref[...],
                         preferred_element_type=jnp.float32) + b_ref[...]


def _matmul(x, w, b, blk=2000):
    n, k = x.shape
    m = w.shape[1]
    return pl.pallas_call(
        _mm_body,
        grid=(n // blk,),
        in_specs=[
            pl.BlockSpec((blk, k), lambda i: (i, 0)),
            pl.BlockSpec((k, m), lambda i: (0, 0)),
            pl.BlockSpec((1, m), lambda i: (0, 0)),
        ],
        out_specs=pl.BlockSpec((blk, m), lambda i: (i, 0)),
        out_shape=jax.ShapeDtypeStruct((n, m), jnp.float32),
    )(x, w, b.reshape(1, -1))


def _leaky(v):
    return jax.nn.leaky_relu(v, 0.01)


def _bn(v, g, b):
    mu = jnp.mean(v, axis=0)
    var = jnp.var(v, axis=0)
    return (v - mu) / jnp.sqrt(var + 1e-5) * g + b


def kernel(x_a, x_b, edge_index_ab, edge_index_ba, W_in, b_in, W_taps,
           b_taps, gamma, beta, W_out, b_out):
    ha = _matmul(x_a, W_in[0], b_in[0])
    hb = _matmul(x_b, W_in[1], b_in[1])
    for l in range(N_LAYERS):
        ya = _leaky(_bn(ha, gamma[l, 0], beta[l, 0]))
        yb = _leaky(_bn(hb, gamma[l, 1], beta[l, 1]))
        za = ya @ W_taps[l, 0, 0] + b_taps[l, 0, 0]
        zb = yb @ W_taps[l, 0, 1] + b_taps[l, 0, 1]
        for i in range(1, N_TAPS + 1):
            na = jax.ops.segment_sum(yb[edge_index_ba[0]], edge_index_ba[1],
                                     num_segments=N)
            nb = jax.ops.segment_sum(ya[edge_index_ab[0]], edge_index_ab[1],
                                     num_segments=N)
            ya, yb = na, nb
            za = za + ya @ W_taps[l, i, 0] + b_taps[l, i, 0]
            zb = zb + yb @ W_taps[l, i, 1] + b_taps[l, i, 1]
        ha = ha + za
        hb = hb + zb
    out_a = _matmul(ha, W_out[0], b_out[0])
    out_b = _matmul(hb, W_out[1], b_out[1])
    return jnp.stack([out_a, out_b])


# speedup vs baseline: 12.7428x; 12.6953x over previous
import functools

import jax
import jax.numpy as jnp
from jax import lax
from jax.experimental import pallas as pl
from jax.experimental.pallas import tpu as pltpu
from jax.experimental.pallas import tpu_sc as plsc

N = 50000
E = 1600000
D_IN = 128
D_H = 32
N_TAPS = 4
N_LAYERS = 2

BLK = 2000
NBLK = N // BLK

NTILES = 16
CH = 100
NB = 8
NROW2D = E // CH
NCH_TILE = NROW2D // NTILES
NGRP = NCH_TILE // NB
YPAD = 50048
RPT = YPAD // NTILES


def _readin_body(xa_ref, xb_ref, w_ref, b_ref, h_ref, sum_ref, sq_ref):
    i = pl.program_id(0)
    ha = jnp.dot(xa_ref[...], w_ref[0], preferred_element_type=jnp.float32) + b_ref[0]
    hb = jnp.dot(xb_ref[...], w_ref[1], preferred_element_type=jnp.float32) + b_ref[1]
    h_ref[0] = ha
    h_ref[1] = hb
    ps = jnp.stack([jnp.sum(ha, 0), jnp.sum(hb, 0)])[:, None, :]
    pq = jnp.stack([jnp.sum(ha * ha, 0), jnp.sum(hb * hb, 0)])[:, None, :]

    @pl.when(i == 0)
    def _():
        sum_ref[...] = ps
        sq_ref[...] = pq

    @pl.when(i > 0)
    def _():
        sum_ref[...] += ps
        sq_ref[...] += pq


def _readin(x_a, x_b, W_in, b_in):
    return pl.pallas_call(
        _readin_body,
        grid=(NBLK,),
        in_specs=[
            pl.BlockSpec((BLK, D_IN), lambda i: (i, 0)),
            pl.BlockSpec((BLK, D_IN), lambda i: (i, 0)),
            pl.BlockSpec((2, D_IN, D_H), lambda i: (0, 0, 0)),
            pl.BlockSpec((2, D_H), lambda i: (0, 0)),
        ],
        out_specs=[
            pl.BlockSpec((2, BLK, D_H), lambda i: (0, i, 0)),
            pl.BlockSpec((2, 1, D_H), lambda i: (0, 0, 0)),
            pl.BlockSpec((2, 1, D_H), lambda i: (0, 0, 0)),
        ],
        out_shape=[
            jax.ShapeDtypeStruct((2, N, D_H), jnp.float32),
            jax.ShapeDtypeStruct((2, 1, D_H), jnp.float32),
            jax.ShapeDtypeStruct((2, 1, D_H), jnp.float32),
        ],
    )(x_a, x_b, W_in, b_in)


def _norm_body(h_ref, sum_ref, sq_ref, g_ref, bt_ref, y_ref):
    mu = sum_ref[...] / N
    var = sq_ref[...] / N - mu * mu
    inv = lax.rsqrt(var + 1e-5)
    v = (h_ref[...] - mu) * inv * g_ref[...] + bt_ref[...]
    y_ref[...] = jnp.where(v >= 0, v, 0.01 * v)


def _norm_leaky(h, ssum, ssq, g, bt):
    return pl.pallas_call(
        _norm_body,
        grid=(NBLK,),
        in_specs=[
            pl.BlockSpec((2, BLK, D_H), lambda i: (0, i, 0)),
            pl.BlockSpec((2, 1, D_H), lambda i: (0, 0, 0)),
            pl.BlockSpec((2, 1, D_H), lambda i: (0, 0, 0)),
            pl.BlockSpec((2, 1, D_H), lambda i: (0, 0, 0)),
            pl.BlockSpec((2, 1, D_H), lambda i: (0, 0, 0)),
        ],
        out_specs=pl.BlockSpec((2, BLK, D_H), lambda i: (0, i, 0)),
        out_shape=jax.ShapeDtypeStruct((2, YPAD, D_H), jnp.float32),
    )(h, ssum, ssq, g[:, None, :], bt[:, None, :])


def _comb_body(h_ref, y0, y1, y2, y3, y4, w_ref, b_ref, ho_ref, sum_ref, sq_ref):
    i = pl.program_id(0)
    acc_a = h_ref[0]
    acc_b = h_ref[1]
    for k, y in enumerate((y0, y1, y2, y3, y4)):
        acc_a = acc_a + jnp.dot(y[0], w_ref[k, 0], preferred_element_type=jnp.float32)
        acc_b = acc_b + jnp.dot(y[1], w_ref[k, 1], preferred_element_type=jnp.float32)
    bsum = jnp.sum(b_ref[...], axis=0)
    acc_a = acc_a + bsum[0]
    acc_b = acc_b + bsum[1]
    ho_ref[0] = acc_a
    ho_ref[1] = acc_b
    ps = jnp.stack([jnp.sum(acc_a, 0), jnp.sum(acc_b, 0)])[:, None, :]
    pq = jnp.stack([jnp.sum(acc_a * acc_a, 0), jnp.sum(acc_b * acc_b, 0)])[:, None, :]

    @pl.when(i == 0)
    def _():
        sum_ref[...] = ps
        sq_ref[...] = pq

    @pl.when(i > 0)
    def _():
        sum_ref[...] += ps
        sq_ref[...] += pq


def _combine(h, ys, Wt, bt):
    yblock = pl.BlockSpec((2, BLK, D_H), lambda i: (0, i, 0))
    return pl.pallas_call(
        _comb_body,
        grid=(NBLK,),
        in_specs=[yblock] * 6 + [
            pl.BlockSpec((N_TAPS + 1, 2, D_H, D_H), lambda i: (0, 0, 0, 0)),
            pl.BlockSpec((N_TAPS + 1, 2, D_H), lambda i: (0, 0, 0)),
        ],
        out_specs=[
            yblock,
            pl.BlockSpec((2, 1, D_H), lambda i: (0, 0, 0)),
            pl.BlockSpec((2, 1, D_H), lambda i: (0, 0, 0)),
        ],
        out_shape=[
            jax.ShapeDtypeStruct((2, N, D_H), jnp.float32),
            jax.ShapeDtypeStruct((2, 1, D_H), jnp.float32),
            jax.ShapeDtypeStruct((2, 1, D_H), jnp.float32),
        ],
    )(h, *ys, Wt, bt)


def _readout_body(h_ref, w_ref, b_ref, o_ref):
    o_ref[0] = jnp.dot(h_ref[0], w_ref[0], preferred_element_type=jnp.float32) + b_ref[0]
    o_ref[1] = jnp.dot(h_ref[1], w_ref[1], preferred_element_type=jnp.float32) + b_ref[1]


def _readout(h, W_out, b_out):
    return pl.pallas_call(
        _readout_body,
        grid=(NBLK,),
        in_specs=[
            pl.BlockSpec((2, BLK, D_H), lambda i: (0, i, 0)),
            pl.BlockSpec((2, D_H, D_IN), lambda i: (0, 0, 0)),
            pl.BlockSpec((2, D_IN), lambda i: (0, 0)),
        ],
        out_specs=pl.BlockSpec((2, BLK, D_IN), lambda i: (0, i, 0)),
        out_shape=jax.ShapeDtypeStruct((2, N, D_IN), jnp.float32),
    )(h, W_out, b_out)


_sc_mesh = plsc.VectorSubcoreMesh(core_axis_name="c", subcore_axis_name="s")


@functools.partial(
    pl.kernel,
    out_type=jax.ShapeDtypeStruct((2 * YPAD, D_H), jnp.float32),
    mesh=_sc_mesh,
    scratch_types=[
        pltpu.VMEM((NB, CH), jnp.int32),
        pltpu.VMEM((NB, CH), jnp.int32),
        pltpu.VMEM((NB, CH, D_H), jnp.float32),
        pltpu.VMEM_SHARED((YPAD, D_H), jnp.float32),
    ] + [pltpu.SemaphoreType.DMA] * NB,
    compiler_params=pltpu.CompilerParams(use_tc_tiling_on_sc=False),
)
def _seg_pair(y2_hbm, srcx_hbm, dstx_hbm, zeros_hbm, out_hbm,
              idx_s, idx_d, rows, acc, *sems):
    c = lax.axis_index("c")
    s = lax.axis_index("s")

    row0 = s * RPT
    pltpu.sync_copy(zeros_hbm.at[pl.ds(row0, RPT), :],
                    acc.at[pl.ds(row0, RPT), :])

    plsc.subcore_barrier()

    crow0 = c * NROW2D + s * NCH_TILE

    @pl.loop(0, NGRP)
    def _(g):
        crow = crow0 + g * NB
        pltpu.sync_copy(srcx_hbm.at[pl.ds(crow, NB), :], idx_s)
        pltpu.sync_copy(dstx_hbm.at[pl.ds(crow, NB), :], idx_d)
        descs = [
            pltpu.async_copy(y2_hbm.at[idx_s.at[b]], rows.at[b], sems[b])
            for b in range(NB)
        ]
        for b in range(NB):
            descs[b].wait()
            pltpu.sync_copy(rows.at[b], acc.at[idx_d.at[b]], add=True)

    plsc.subcore_barrier()

    pltpu.sync_copy(acc.at[pl.ds(row0, RPT), :],
                    out_hbm.at[pl.ds(c * YPAD + row0, RPT), :])


def _seg_tap(y2, srcx, dstx, zeros):
    return _seg_pair(y2, srcx, dstx, zeros)


def kernel(x_a, x_b, edge_index_ab, edge_index_ba, W_in, b_in, W_taps,
           b_taps, gamma, beta, W_out, b_out):
    src_a = edge_index_ba[0].reshape(NROW2D, CH) + YPAD
    dst_a = edge_index_ba[1].reshape(NROW2D, CH)
    src_b = edge_index_ab[0].reshape(NROW2D, CH)
    dst_b = edge_index_ab[1].reshape(NROW2D, CH)
    srcx = jnp.concatenate([src_a, src_b])
    dstx = jnp.concatenate([dst_a, dst_b])
    zeros = jnp.zeros((YPAD, D_H), jnp.float32)

    h, ssum, ssq = _readin(x_a, x_b, W_in, b_in)
    for l in range(N_LAYERS):
        y = _norm_leaky(h, ssum, ssq, gamma[l], beta[l])
        ys = [y]
        y2 = y.reshape(2 * YPAD, D_H)
        for _ in range(N_TAPS):
            y2 = _seg_tap(y2, srcx, dstx, zeros)
            ys.append(y2.reshape(2, YPAD, D_H))
        h, ssum, ssq = _combine(h, ys, W_taps[l], b_taps[l])
    return _readout(h, W_out, b_out)


# speedup vs baseline: 17.7968x; 1.3966x over previous
import functools

import jax
import jax.numpy as jnp
from jax import lax
from jax.experimental import pallas as pl
from jax.experimental.pallas import tpu as pltpu
from jax.experimental.pallas import tpu_sc as plsc

N = 50000
E = 1600000
D_IN = 128
D_H = 32
N_TAPS = 4
N_LAYERS = 2

BLK = 2000
NBLK = N // BLK

NTILES = 16
CH = 100
NB = 8
NROW2D = E // CH
NCH_TILE = NROW2D // NTILES
NGRP = NCH_TILE // NB
YPAD = 50048
RPT = YPAD // NTILES


def _readin_body(xa_ref, xb_ref, w_ref, b_ref, h_ref, sum_ref, sq_ref):
    i = pl.program_id(0)
    ha = jnp.dot(xa_ref[...], w_ref[0], preferred_element_type=jnp.float32) + b_ref[0]
    hb = jnp.dot(xb_ref[...], w_ref[1], preferred_element_type=jnp.float32) + b_ref[1]
    h_ref[0] = ha
    h_ref[1] = hb
    ps = jnp.stack([jnp.sum(ha, 0), jnp.sum(hb, 0)])[:, None, :]
    pq = jnp.stack([jnp.sum(ha * ha, 0), jnp.sum(hb * hb, 0)])[:, None, :]

    @pl.when(i == 0)
    def _():
        sum_ref[...] = ps
        sq_ref[...] = pq

    @pl.when(i > 0)
    def _():
        sum_ref[...] += ps
        sq_ref[...] += pq


def _readin(x_a, x_b, W_in, b_in):
    return pl.pallas_call(
        _readin_body,
        grid=(NBLK,),
        in_specs=[
            pl.BlockSpec((BLK, D_IN), lambda i: (i, 0)),
            pl.BlockSpec((BLK, D_IN), lambda i: (i, 0)),
            pl.BlockSpec((2, D_IN, D_H), lambda i: (0, 0, 0)),
            pl.BlockSpec((2, D_H), lambda i: (0, 0)),
        ],
        out_specs=[
            pl.BlockSpec((2, BLK, D_H), lambda i: (0, i, 0)),
            pl.BlockSpec((2, 1, D_H), lambda i: (0, 0, 0)),
            pl.BlockSpec((2, 1, D_H), lambda i: (0, 0, 0)),
        ],
        out_shape=[
            jax.ShapeDtypeStruct((2, N, D_H), jnp.float32),
            jax.ShapeDtypeStruct((2, 1, D_H), jnp.float32),
            jax.ShapeDtypeStruct((2, 1, D_H), jnp.float32),
        ],
    )(x_a, x_b, W_in, b_in)


def _norm_body(h_ref, sum_ref, sq_ref, g_ref, bt_ref, y_ref):
    mu = sum_ref[...] / N
    var = sq_ref[...] / N - mu * mu
    inv = lax.rsqrt(var + 1e-5)
    v = (h_ref[...] - mu) * inv * g_ref[...] + bt_ref[...]
    y_ref[...] = jnp.where(v >= 0, v, 0.01 * v)


def _norm_leaky(h, ssum, ssq, g, bt):
    return pl.pallas_call(
        _norm_body,
        grid=(NBLK,),
        in_specs=[
            pl.BlockSpec((2, BLK, D_H), lambda i: (0, i, 0)),
            pl.BlockSpec((2, 1, D_H), lambda i: (0, 0, 0)),
            pl.BlockSpec((2, 1, D_H), lambda i: (0, 0, 0)),
            pl.BlockSpec((2, 1, D_H), lambda i: (0, 0, 0)),
            pl.BlockSpec((2, 1, D_H), lambda i: (0, 0, 0)),
        ],
        out_specs=pl.BlockSpec((2, BLK, D_H), lambda i: (0, i, 0)),
        out_shape=jax.ShapeDtypeStruct((2, YPAD, D_H), jnp.float32),
    )(h, ssum, ssq, g[:, None, :], bt[:, None, :])


def _comb_body(h_ref, y0, y1, y2, y3, y4, w_ref, b_ref, ho_ref, sum_ref, sq_ref):
    i = pl.program_id(0)
    acc_a = h_ref[0]
    acc_b = h_ref[1]
    for k, y in enumerate((y0, y1, y2, y3, y4)):
        acc_a = acc_a + jnp.dot(y[0], w_ref[k, 0], preferred_element_type=jnp.float32)
        acc_b = acc_b + jnp.dot(y[1], w_ref[k, 1], preferred_element_type=jnp.float32)
    bsum = jnp.sum(b_ref[...], axis=0)
    acc_a = acc_a + bsum[0]
    acc_b = acc_b + bsum[1]
    ho_ref[0] = acc_a
    ho_ref[1] = acc_b
    ps = jnp.stack([jnp.sum(acc_a, 0), jnp.sum(acc_b, 0)])[:, None, :]
    pq = jnp.stack([jnp.sum(acc_a * acc_a, 0), jnp.sum(acc_b * acc_b, 0)])[:, None, :]

    @pl.when(i == 0)
    def _():
        sum_ref[...] = ps
        sq_ref[...] = pq

    @pl.when(i > 0)
    def _():
        sum_ref[...] += ps
        sq_ref[...] += pq


def _combine(h, ys, Wt, bt):
    yblock = pl.BlockSpec((2, BLK, D_H), lambda i: (0, i, 0))
    return pl.pallas_call(
        _comb_body,
        grid=(NBLK,),
        in_specs=[yblock] * 6 + [
            pl.BlockSpec((N_TAPS + 1, 2, D_H, D_H), lambda i: (0, 0, 0, 0)),
            pl.BlockSpec((N_TAPS + 1, 2, D_H), lambda i: (0, 0, 0)),
        ],
        out_specs=[
            yblock,
            pl.BlockSpec((2, 1, D_H), lambda i: (0, 0, 0)),
            pl.BlockSpec((2, 1, D_H), lambda i: (0, 0, 0)),
        ],
        out_shape=[
            jax.ShapeDtypeStruct((2, N, D_H), jnp.float32),
            jax.ShapeDtypeStruct((2, 1, D_H), jnp.float32),
            jax.ShapeDtypeStruct((2, 1, D_H), jnp.float32),
        ],
    )(h, *ys, Wt, bt)


def _readout_body(h_ref, w_ref, b_ref, o_ref):
    o_ref[0] = jnp.dot(h_ref[0], w_ref[0], preferred_element_type=jnp.float32) + b_ref[0]
    o_ref[1] = jnp.dot(h_ref[1], w_ref[1], preferred_element_type=jnp.float32) + b_ref[1]


def _readout(h, W_out, b_out):
    return pl.pallas_call(
        _readout_body,
        grid=(NBLK,),
        in_specs=[
            pl.BlockSpec((2, BLK, D_H), lambda i: (0, i, 0)),
            pl.BlockSpec((2, D_H, D_IN), lambda i: (0, 0, 0)),
            pl.BlockSpec((2, D_IN), lambda i: (0, 0)),
        ],
        out_specs=pl.BlockSpec((2, BLK, D_IN), lambda i: (0, i, 0)),
        out_shape=jax.ShapeDtypeStruct((2, N, D_IN), jnp.float32),
    )(h, W_out, b_out)


_sc_mesh = plsc.VectorSubcoreMesh(core_axis_name="c", subcore_axis_name="s")


@functools.partial(
    pl.kernel,
    out_type=jax.ShapeDtypeStruct((2 * YPAD, D_H), jnp.float32),
    mesh=_sc_mesh,
    scratch_types=[
        pltpu.VMEM((2, NB, 2, CH), jnp.int32),
        pltpu.VMEM((NB, CH, D_H), jnp.float32),
        pltpu.VMEM_SHARED((YPAD, D_H), jnp.float32),
    ] + [pltpu.SemaphoreType.DMA] * (2 * NB + 1),
    compiler_params=pltpu.CompilerParams(use_tc_tiling_on_sc=False),
)
def _seg_pair(y2_hbm, sdx_hbm, zeros_hbm, out_hbm, idx, rows, acc, *sems):
    gsem = sems[:NB]
    ssem = sems[NB:2 * NB]
    isem = sems[2 * NB]
    c = lax.axis_index("c")
    s = lax.axis_index("s")
    row0 = s * RPT
    crow0 = c * NROW2D + s * NCH_TILE

    def idx_copy(g):
        p = g & 1
        return pltpu.make_async_copy(
            sdx_hbm.at[pl.ds(crow0 + g * NB, NB), :, :], idx.at[p], isem)

    def fire_gathers(g):
        p = g & 1
        for b in range(NB):
            pltpu.async_copy(y2_hbm.at[idx.at[p, b, 0]], rows.at[b], gsem[b])

    def wait_and_scatter(g):
        p = g & 1
        for b in range(NB):
            pltpu.make_async_copy(
                y2_hbm.at[idx.at[p, b, 0]], rows.at[b], gsem[b]).wait()
            pltpu.async_copy(rows.at[b], acc.at[idx.at[p, b, 1]], ssem[b],
                             add=True)

    def drain_scatters(g):
        p = g & 1
        for b in range(NB):
            pltpu.make_async_copy(
                rows.at[b], acc.at[idx.at[p, b, 1]], ssem[b]).wait()

    idx_copy(0).start()
    pltpu.sync_copy(zeros_hbm.at[pl.ds(row0, RPT), :],
                    acc.at[pl.ds(row0, RPT), :])
    plsc.subcore_barrier()

    idx_copy(0).wait()
    idx_copy(1).start()
    fire_gathers(0)
    wait_and_scatter(0)

    @pl.loop(1, NGRP - 1)
    def _(g):
        drain_scatters(g - 1)
        idx_copy(g).wait()
        idx_copy(g + 1).start()
        fire_gathers(g)
        wait_and_scatter(g)

    drain_scatters(NGRP - 2)
    idx_copy(NGRP - 1).wait()
    fire_gathers(NGRP - 1)
    wait_and_scatter(NGRP - 1)
    drain_scatters(NGRP - 1)

    plsc.subcore_barrier()

    pltpu.sync_copy(acc.at[pl.ds(row0, RPT), :],
                    out_hbm.at[pl.ds(c * YPAD + row0, RPT), :])


def _seg_tap(y2, sdx, zeros):
    return _seg_pair(y2, sdx, zeros)


def kernel(x_a, x_b, edge_index_ab, edge_index_ba, W_in, b_in, W_taps,
           b_taps, gamma, beta, W_out, b_out):
    src_a = edge_index_ba[0].reshape(NROW2D, CH) + YPAD
    dst_a = edge_index_ba[1].reshape(NROW2D, CH)
    src_b = edge_index_ab[0].reshape(NROW2D, CH)
    dst_b = edge_index_ab[1].reshape(NROW2D, CH)
    srcx = jnp.concatenate([src_a, src_b])
    dstx = jnp.concatenate([dst_a, dst_b])
    sdx = jnp.stack([srcx, dstx], axis=1)
    zeros = jnp.zeros((YPAD, D_H), jnp.float32)

    h, ssum, ssq = _readin(x_a, x_b, W_in, b_in)
    for l in range(N_LAYERS):
        y = _norm_leaky(h, ssum, ssq, gamma[l], beta[l])
        ys = [y]
        y2 = y.reshape(2 * YPAD, D_H)
        for _ in range(N_TAPS):
            y2 = _seg_tap(y2, sdx, zeros)
            ys.append(y2.reshape(2, YPAD, D_H))
        h, ssum, ssq = _combine(h, ys, W_taps[l], b_taps[l])
    return _readout(h, W_out, b_out)
